# baseline (speedup 1.0000x reference)
import functools

import jax
import jax.numpy as jnp
from jax import lax
from jax.experimental import pallas as pl
from jax.experimental.pallas import tpu as pltpu
from jax.experimental.pallas import tpu_sc as plsc

B = 16384
N_DENSE = 13
N_SPARSE = 26
VOCAB = 100000
VPAD = 100352
EMB = 16
NCOL = N_DENSE + N_SPARSE

NC = 2
NS = 16
NW = NC * NS
L = 16

ROWS_PER_TILE = B // NW
X_PER_TILE = ROWS_PER_TILE * NCOL
CHUNK = 128
CBLK = ROWS_PER_TILE // CHUNK
VREG_PER_F = ROWS_PER_TILE // L

PHASES = ((0, 13), (13, 26))


def _table_sum(tables_t, lo, n):
  def body(t_ref, s_ref):
    s_ref[pl.ds(0, VOCAB)] = jnp.sum(t_ref[...], axis=(0, 1))

  return pl.pallas_call(
      body,
      grid=(n,),
      in_specs=[pl.BlockSpec((1, EMB, VOCAB), lambda f: (f + lo, 0, 0))],
      out_specs=pl.BlockSpec((VPAD,), lambda f: (f,)),
      out_shape=jax.ShapeDtypeStruct((n * VPAD,), jnp.float32),
  )(tables_t)


def _tc_prep(x_t, weight):
  def body(x_ref, w_ref, o_dense, *o_refs):
    x = x_ref[...]
    dense = jax.lax.dot_general(
        w_ref[...], x[:N_DENSE, :],
        dimension_numbers=(((0,), (0,)), ((), ())),
        preferred_element_type=jnp.float32,
    )
    o_dense[...] = dense[0]
    for p, (lo, hi) in enumerate(PHASES):
      n = hi - lo
      for fl in range(n):
        row = x[N_DENSE + lo + fl].astype(jnp.int32) + fl * VPAD
        for w in range(NW):
          o_refs[p][pl.ds((w * n + fl) * ROWS_PER_TILE, ROWS_PER_TILE)] = (
              row[w * ROWS_PER_TILE:(w + 1) * ROWS_PER_TILE])

  return pl.pallas_call(
      body,
      in_specs=[
          pl.BlockSpec((NCOL, B), lambda: (0, 0)),
          pl.BlockSpec((N_DENSE, 1), lambda: (0, 0)),
      ],
      out_specs=[pl.BlockSpec((B,), lambda: (0,))] + [
          pl.BlockSpec((B * (hi - lo),), lambda: (0,)) for lo, hi in PHASES
      ],
      out_shape=[jax.ShapeDtypeStruct((B,), jnp.float32)] + [
          jax.ShapeDtypeStruct((B * (hi - lo),), jnp.int32)
          for lo, hi in PHASES
      ],
  )(x_t, weight)


def _sc_phase(idx_flat, s_p, n, extras=None):
  final = extras is not None
  n_extra = len(extras) if final else 0
  g_per_tile = ROWS_PER_TILE * n
  mesh = plsc.VectorSubcoreMesh(core_axis_name="c", subcore_axis_name="s")

  s_len = n * VPAD
  s_per_tile = s_len // NS
  scratch = [
      pltpu.VMEM((n * CBLK, CHUNK), jnp.int32),
      pltpu.VMEM((2, n, CHUNK), jnp.float32),
      pltpu.VMEM((ROWS_PER_TILE,), jnp.float32),
      pltpu.VMEM_SHARED((s_len,), jnp.float32),
      pltpu.SemaphoreType.DMA,
      pltpu.SemaphoreType.DMA,
      pltpu.SemaphoreType.DMA,
  ]
  if final:
    scratch.insert(3, pltpu.VMEM((n_extra, ROWS_PER_TILE), jnp.float32))

  @functools.partial(
      pl.kernel,
      out_type=jax.ShapeDtypeStruct((B,), jnp.float32),
      mesh=mesh,
      scratch_types=scratch,
      compiler_params=pltpu.CompilerParams(
          use_tc_tiling_on_sc=False, needs_layout_passes=False),
  )
  def k(*refs):
    if final:
      (idx_hbm, s_hbm), e_hbms = refs[:2], refs[2:2 + n_extra]
      (out_hbm, idxb, gbuf, outb, eb, ssh, semi, sem0, sem1) = (
          refs[2 + n_extra:])
    else:
      (idx_hbm, s_hbm, out_hbm, idxb, gbuf, outb, ssh, semi, sem0, sem1) = refs
    sid = lax.axis_index("s")
    wid = sid * NC + lax.axis_index("c")
    base_b = wid * ROWS_PER_TILE

    scp = pltpu.async_copy(
        s_hbm.at[pl.ds(sid * s_per_tile, s_per_tile)],
        ssh.at[pl.ds(sid * s_per_tile, s_per_tile)], sem0)
    stage = [
        pltpu.async_copy(
            idx_hbm.at[pl.ds(wid * g_per_tile + r * CHUNK, CHUNK)],
            idxb.at[r], semi)
        for r in range(n * CBLK)
    ]
    ecps = []
    if final:
      for i, e in enumerate(e_hbms):
        ecps.append(pltpu.async_copy(
            e.at[pl.ds(base_b, ROWS_PER_TILE)], eb.at[i], semi))
    for c in stage:
      c.wait()
    scp.wait()
    plsc.subcore_barrier()

    def fire(cc, p, sem):
      return [
          pltpu.async_copy(
              ssh.at[idxb.at[f * CBLK + cc]],
              gbuf.at[p, f],
              sem,
          ) for f in range(n)
      ]

    def reduce(cc, p):
      for g in range(CHUNK // L):
        vs = [gbuf[p, f, pl.ds(g * L, L)] for f in range(min(4, n))]
        for f in range(4, n, 4):
          for j in range(4):
            if f + j < n:
              vs[j] = vs[j] + gbuf[p, f + j, pl.ds(g * L, L)]
        while len(vs) > 1:
          vs = [vs[i] + vs[i + 1] for i in range(0, len(vs) - 1, 2)] + (
              [vs[-1]] if len(vs) % 2 else [])
        outb[pl.ds(cc * CHUNK + g * L, L)] = vs[0]

    sems = [sem0, sem1]
    copies = fire(0, 0, sems[0])
    for cc in range(CBLK):
      if cc + 1 < CBLK:
        nxt = fire(cc + 1, (cc + 1) % 2, sems[(cc + 1) % 2])
      for c in copies:
        c.wait()
      reduce(cc, cc % 2)
      if cc + 1 < CBLK:
        copies = nxt

    if final:
      for c in ecps:
        c.wait()
      one = jnp.full((L,), 1.0, jnp.float32)
      for t in range(ROWS_PER_TILE // L):
        sl = pl.ds(t * L, L)
        z = outb[sl]
        for i in range(n_extra):
          z = z + eb[i, sl]
        outb[sl] = one / (one + jnp.exp(-z))

    pltpu.sync_copy(outb, out_hbm.at[pl.ds(base_b, ROWS_PER_TILE)])

  args = (idx_flat, s_p) + (tuple(extras) if final else ())
  return k(*args)


@jax.jit
def kernel(X, tables, weight):
  x_t = X.T
  tables_t = jnp.transpose(tables, (0, 2, 1))
  dense, *idx_phases = _tc_prep(x_t, weight)
  partials = []
  for p, (lo, hi) in enumerate(PHASES[:-1]):
    s_p = _table_sum(tables_t, lo, hi - lo)
    partials.append(_sc_phase(idx_phases[p], s_p, hi - lo))
  lo, hi = PHASES[-1]
  s_p = _table_sum(tables_t, lo, hi - lo)
  out = _sc_phase(idx_phases[-1], s_p, hi - lo,
                  extras=tuple(partials) + (dense,))
  return out.reshape(B, 1)


# speedup vs baseline: 1.0158x; 1.0158x over previous
import functools

import jax
import jax.numpy as jnp
from jax import lax
from jax.experimental import pallas as pl
from jax.experimental.pallas import tpu as pltpu
from jax.experimental.pallas import tpu_sc as plsc

B = 16384
N_DENSE = 13
N_SPARSE = 26
VOCAB = 100000
VPAD = 100352
EMB = 16
NCOL = N_DENSE + N_SPARSE

NC = 2
NS = 16
NW = NC * NS
L = 16

ROWS_PER_TILE = B // NW
CHUNK = 128
CBLK = ROWS_PER_TILE // CHUNK

PHASES = ((0, 11), (11, 21), (21, 26))


def _table_sum(tables_t, lo, n):
  def body(t_ref, s_ref):
    s_ref[pl.ds(0, VOCAB)] = jnp.sum(t_ref[...], axis=(0, 1))

  return pl.pallas_call(
      body,
      grid=(n,),
      in_specs=[pl.BlockSpec((1, EMB, VOCAB), lambda f: (f + lo, 0, 0))],
      out_specs=pl.BlockSpec((VPAD,), lambda f: (f,)),
      out_shape=jax.ShapeDtypeStruct((n * VPAD,), jnp.float32),
  )(tables_t)


def _tc_prep(x_t, weight):
  def body(x_ref, w_ref, o_dense, *o_refs):
    x = x_ref[...]
    dense = jax.lax.dot_general(
        w_ref[...], x[:N_DENSE, :],
        dimension_numbers=(((0,), (0,)), ((), ())),
        preferred_element_type=jnp.float32,
    )
    o_dense[...] = dense[0]
    for p, (lo, hi) in enumerate(PHASES):
      n = hi - lo
      for fl in range(n):
        row = x[N_DENSE + lo + fl].astype(jnp.int32) + fl * VPAD
        for w in range(NW):
          o_refs[p][pl.ds((w * n + fl) * ROWS_PER_TILE, ROWS_PER_TILE)] = (
              row[w * ROWS_PER_TILE:(w + 1) * ROWS_PER_TILE])

  return pl.pallas_call(
      body,
      in_specs=[
          pl.BlockSpec((NCOL, B), lambda: (0, 0)),
          pl.BlockSpec((N_DENSE, 1), lambda: (0, 0)),
      ],
      out_specs=[pl.BlockSpec((B,), lambda: (0,))] + [
          pl.BlockSpec((B * (hi - lo),), lambda: (0,)) for lo, hi in PHASES
      ],
      out_shape=[jax.ShapeDtypeStruct((B,), jnp.float32)] + [
          jax.ShapeDtypeStruct((B * (hi - lo),), jnp.int32)
          for lo, hi in PHASES
      ],
  )(x_t, weight)


def _sc_phase(idx_flat, s_p, n, extras=None):
  final = extras is not None
  n_extra = len(extras) if final else 0
  g_per_tile = ROWS_PER_TILE * n
  mesh = plsc.VectorSubcoreMesh(core_axis_name="c", subcore_axis_name="s")

  s_len = n * VPAD
  s_per_tile = s_len // NS
  scratch = [
      pltpu.VMEM((n * CBLK, CHUNK), jnp.int32),
      pltpu.VMEM((2, n, CHUNK), jnp.float32),
      pltpu.VMEM((ROWS_PER_TILE,), jnp.float32),
      pltpu.VMEM_SHARED((s_len,), jnp.float32),
      pltpu.SemaphoreType.DMA,
      pltpu.SemaphoreType.DMA,
      pltpu.SemaphoreType.DMA,
  ]
  if final:
    scratch.insert(3, pltpu.VMEM((n_extra, ROWS_PER_TILE), jnp.float32))

  @functools.partial(
      pl.kernel,
      out_type=jax.ShapeDtypeStruct((B,), jnp.float32),
      mesh=mesh,
      scratch_types=scratch,
      compiler_params=pltpu.CompilerParams(
          use_tc_tiling_on_sc=False, needs_layout_passes=False),
  )
  def k(*refs):
    if final:
      (idx_hbm, s_hbm), e_hbms = refs[:2], refs[2:2 + n_extra]
      (out_hbm, idxb, gbuf, outb, eb, ssh, semi, sem0, sem1) = (
          refs[2 + n_extra:])
    else:
      (idx_hbm, s_hbm, out_hbm, idxb, gbuf, outb, ssh, semi, sem0, sem1) = refs
    sid = lax.axis_index("s")
    wid = sid * NC + lax.axis_index("c")
    base_b = wid * ROWS_PER_TILE

    scp = pltpu.async_copy(
        s_hbm.at[pl.ds(sid * s_per_tile, s_per_tile)],
        ssh.at[pl.ds(sid * s_per_tile, s_per_tile)], sem0)
    stage = [
        pltpu.async_copy(
            idx_hbm.at[pl.ds(wid * g_per_tile + r * CHUNK, CHUNK)],
            idxb.at[r], semi)
        for r in range(n * CBLK)
    ]
    ecps = []
    if final:
      for i, e in enumerate(e_hbms):
        ecps.append(pltpu.async_copy(
            e.at[pl.ds(base_b, ROWS_PER_TILE)], eb.at[i], semi))
    for c in stage:
      c.wait()
    scp.wait()
    plsc.subcore_barrier()

    def fire(cc, p, sem):
      return [
          pltpu.async_copy(
              ssh.at[idxb.at[f * CBLK + cc]],
              gbuf.at[p, f],
              sem,
          ) for f in range(n)
      ]

    def reduce(cc, p):
      for g in range(CHUNK // L):
        vs = [gbuf[p, f, pl.ds(g * L, L)] for f in range(min(4, n))]
        for f in range(4, n, 4):
          for j in range(4):
            if f + j < n:
              vs[j] = vs[j] + gbuf[p, f + j, pl.ds(g * L, L)]
        while len(vs) > 1:
          vs = [vs[i] + vs[i + 1] for i in range(0, len(vs) - 1, 2)] + (
              [vs[-1]] if len(vs) % 2 else [])
        outb[pl.ds(cc * CHUNK + g * L, L)] = vs[0]

    sems = [sem0, sem1]
    copies = fire(0, 0, sems[0])
    for cc in range(CBLK):
      if cc + 1 < CBLK:
        nxt = fire(cc + 1, (cc + 1) % 2, sems[(cc + 1) % 2])
      for c in copies:
        c.wait()
      reduce(cc, cc % 2)
      if cc + 1 < CBLK:
        copies = nxt

    if final:
      for c in ecps:
        c.wait()
      one = jnp.full((L,), 1.0, jnp.float32)
      for t in range(ROWS_PER_TILE // L):
        sl = pl.ds(t * L, L)
        z = outb[sl]
        for i in range(n_extra):
          z = z + eb[i, sl]
        outb[sl] = one / (one + jnp.exp(-z))

    pltpu.sync_copy(outb, out_hbm.at[pl.ds(base_b, ROWS_PER_TILE)])

  args = (idx_flat, s_p) + (tuple(extras) if final else ())
  return k(*args)


@jax.jit
def kernel(X, tables, weight):
  x_t = X.T
  tables_t = jnp.transpose(tables, (0, 2, 1))
  dense, *idx_phases = _tc_prep(x_t, weight)
  partials = []
  for p, (lo, hi) in enumerate(PHASES[:-1]):
    s_p = _table_sum(tables_t, lo, hi - lo)
    partials.append(_sc_phase(idx_phases[p], s_p, hi - lo))
  lo, hi = PHASES[-1]
  s_p = _table_sum(tables_t, lo, hi - lo)
  out = _sc_phase(idx_phases[-1], s_p, hi - lo,
                  extras=tuple(partials) + (dense,))
  return out.reshape(B, 1)
